# baseline (speedup 1.0000x reference)
import functools

import jax
import jax.numpy as jnp
from jax.experimental import pallas as pl
from jax.experimental.pallas import tpu as pltpu

N, C, H, W = 32, 192, 56, 56
G = 8
EPS = 1e-05
CB = 4
HW = H * W


WP = 72


def _conv_pool_kernel(x_ref, w2_ref, y_ref):
    xb = x_ref[...]
    xt = jnp.transpose(xb).reshape(CB, H, W, N)
    zw = jnp.zeros((CB, H, 8, N), xb.dtype)
    xq = jnp.concatenate([zw, xt, zw], axis=2)
    zh = jnp.zeros((CB, 1, WP, N), xb.dtype)
    xp = jnp.concatenate([zh, xq, zh], axis=1)
    ph = jnp.concatenate([xp[:, kh:kh + H, :, :] for kh in range(3)], axis=-1)
    ph2 = ph.reshape(CB * H * WP, 96)
    z = jax.lax.dot_general(ph2, w2_ref[...], (((1,), (0,)), ((), ())),
                            preferred_element_type=jnp.float32)
    z = z.reshape(CB, H, WP, 96)
    acc = (z[:, :, 7:7 + W, 0:32]
           + z[:, :, 8:8 + W, 32:64]
           + z[:, :, 9:9 + W, 64:96])
    r = jnp.maximum(acc, 0.0)
    y_ref[0] = jnp.sum(r, axis=(1, 2)) * jnp.float32(1.0 / HW)


def _router_kernel(y_ref, w1t_ref, b1_ref, w2t_ref, b2_ref, oh_ref):
    y = y_ref[...]
    h1 = jnp.dot(y, w1t_ref[...],
                 preferred_element_type=jnp.float32) + b1_ref[...]
    h2 = jnp.dot(h1, w2t_ref[...],
                 preferred_element_type=jnp.float32) + b2_ref[...]
    m = jnp.max(h2, axis=1, keepdims=True)
    idx = jax.lax.broadcasted_iota(jnp.int32, (C, G), 1)
    first = jnp.min(jnp.where(h2 >= m, idx, G), axis=1, keepdims=True)
    oh_ref[...] = (idx == first).astype(jnp.float32)


def _norm_kernel(x_ref, oh_ref, g_ref, b_ref, o_ref):
    xb = x_ref[0]
    oh = oh_ref[...]
    rs = jnp.sum(xb, axis=1, keepdims=True)
    rs2 = jnp.sum(xb * xb, axis=1, keepdims=True)
    ones = jnp.ones((C, 1), jnp.float32)
    cdot = functools.partial(jax.lax.dot_general,
                             dimension_numbers=(((0,), (0,)), ((), ())),
                             preferred_element_type=jnp.float32)
    gsum = cdot(oh, rs)
    gsum2 = cdot(oh, rs2)
    n = cdot(oh, ones) * jnp.float32(HW)
    mean = gsum / jnp.maximum(n, 1.0)
    var = (gsum2 - n * mean * mean) / jnp.maximum(n - 1.0, 1.0)
    inv = 1.0 / jnp.sqrt(var + EPS)
    mean_c = jnp.dot(oh, mean, preferred_element_type=jnp.float32)
    inv_c = jnp.dot(oh, inv, preferred_element_type=jnp.float32)
    a = inv_c * g_ref[...]
    o_ref[0] = (xb - mean_c) * a + b_ref[...]


def kernel(x, conv_w, fc1_w, fc1_b, fc2_w, fc2_b, gamma, beta):
    xf = x.reshape(N, C * HW)
    w2 = jnp.transpose(conv_w, (2, 1, 3, 0)).reshape(96, 96)

    y = pl.pallas_call(
        _conv_pool_kernel,
        grid=(C // CB,),
        in_specs=[
            pl.BlockSpec((N, CB * HW), lambda i: (0, i)),
            pl.BlockSpec((96, 96), lambda i: (0, 0)),
        ],
        out_specs=pl.BlockSpec((1, CB, N), lambda i: (i, 0, 0)),
        out_shape=jax.ShapeDtypeStruct((C // CB, CB, N), jnp.float32),
        compiler_params=pltpu.CompilerParams(
            dimension_semantics=("arbitrary",)),
    )(xf, w2)
    y = y.reshape(C, N)

    onehot = pl.pallas_call(
        _router_kernel,
        out_shape=jax.ShapeDtypeStruct((C, G), jnp.float32),
    )(y, fc1_w.T, fc1_b.reshape(1, N), fc2_w.T, fc2_b.reshape(1, G))

    x2 = x.reshape(N, C, HW)
    out = pl.pallas_call(
        _norm_kernel,
        grid=(N,),
        in_specs=[
            pl.BlockSpec((1, C, HW), lambda n: (n, 0, 0)),
            pl.BlockSpec((C, G), lambda n: (0, 0)),
            pl.BlockSpec((C, 1), lambda n: (0, 0)),
            pl.BlockSpec((C, 1), lambda n: (0, 0)),
        ],
        out_specs=pl.BlockSpec((1, C, HW), lambda n: (n, 0, 0)),
        out_shape=jax.ShapeDtypeStruct((N, C, HW), jnp.float32),
        compiler_params=pltpu.CompilerParams(
            dimension_semantics=("parallel",)),
    )(x2, onehot, gamma.reshape(C, 1), beta.reshape(C, 1))
    return out.reshape(N, C, H, W)


# speedup vs baseline: 1.0447x; 1.0447x over previous
import functools

import jax
import jax.numpy as jnp
from jax.experimental import pallas as pl
from jax.experimental.pallas import tpu as pltpu

N, C, H, W = 32, 192, 56, 56
G = 8
EPS = 1e-05
CB = 4
HW = H * W
WP = 64


def _conv_pool_kernel(x_ref, w2_ref, y_ref):
    xb = x_ref[...]
    xt = jnp.transpose(xb, (1, 2, 3, 0))
    zw0 = jnp.zeros((CB, H, 1, N), xb.dtype)
    zw1 = jnp.zeros((CB, H, WP - W - 1, N), xb.dtype)
    xq = jnp.concatenate([zw0, xt, zw1], axis=2)
    zh = jnp.zeros((CB, 1, WP, N), xb.dtype)
    xp = jnp.concatenate([zh, xq, zh], axis=1)
    ph = jnp.concatenate([xp[:, kh:kh + H, :, :] for kh in range(3)], axis=-1)
    ph2 = ph.reshape(CB * H * WP, 96)
    z = jax.lax.dot_general(ph2, w2_ref[...], (((1,), (0,)), ((), ())),
                            preferred_element_type=jnp.float32)
    z = z.reshape(CB, H, WP, 96)
    acc = (z[:, :, 0:W, 0:32]
           + z[:, :, 1:W + 1, 32:64]
           + z[:, :, 2:W + 2, 64:96])
    r = jnp.maximum(acc, 0.0)
    y_ref[0] = jnp.sum(r, axis=(1, 2)) * jnp.float32(1.0 / HW)


def _router_kernel(y_ref, w1t_ref, b1_ref, w2t_ref, b2_ref, oh_ref):
    y = y_ref[...]
    h1 = jnp.dot(y, w1t_ref[...],
                 preferred_element_type=jnp.float32) + b1_ref[...]
    h2 = jnp.dot(h1, w2t_ref[...],
                 preferred_element_type=jnp.float32) + b2_ref[...]
    m = jnp.max(h2, axis=1, keepdims=True)
    idx = jax.lax.broadcasted_iota(jnp.int32, (C, G), 1)
    first = jnp.min(jnp.where(h2 >= m, idx, G), axis=1, keepdims=True)
    oh_ref[...] = (idx == first).astype(jnp.float32)


def _norm_kernel(x_ref, oh_ref, g_ref, b_ref, o_ref):
    xb = x_ref[0]
    oh = oh_ref[...]
    rs = jnp.sum(xb, axis=(1, 2)).reshape(C, 1)
    rs2 = jnp.sum(xb * xb, axis=(1, 2)).reshape(C, 1)
    ones = jnp.ones((C, 1), jnp.float32)
    cdot = functools.partial(jax.lax.dot_general,
                             dimension_numbers=(((0,), (0,)), ((), ())),
                             preferred_element_type=jnp.float32)
    gsum = cdot(oh, rs)
    gsum2 = cdot(oh, rs2)
    n = cdot(oh, ones) * jnp.float32(HW)
    mean = gsum / jnp.maximum(n, 1.0)
    var = (gsum2 - n * mean * mean) / jnp.maximum(n - 1.0, 1.0)
    inv = 1.0 / jnp.sqrt(var + EPS)
    mean_c = jnp.dot(oh, mean, preferred_element_type=jnp.float32)
    inv_c = jnp.dot(oh, inv, preferred_element_type=jnp.float32)
    a = (inv_c * g_ref[...])[:, :, None]
    o_ref[0] = (xb - mean_c[:, :, None]) * a + b_ref[...][:, :, None]


def kernel(x, conv_w, fc1_w, fc1_b, fc2_w, fc2_b, gamma, beta):
    w2 = jnp.transpose(conv_w, (2, 1, 3, 0)).reshape(96, 96)

    y = pl.pallas_call(
        _conv_pool_kernel,
        grid=(C // CB,),
        in_specs=[
            pl.BlockSpec((N, CB, H, W), lambda i: (0, i, 0, 0)),
            pl.BlockSpec((96, 96), lambda i: (0, 0)),
        ],
        out_specs=pl.BlockSpec((1, CB, N), lambda i: (i, 0, 0)),
        out_shape=jax.ShapeDtypeStruct((C // CB, CB, N), jnp.float32),
        compiler_params=pltpu.CompilerParams(
            dimension_semantics=("arbitrary",)),
    )(x, w2)
    y = y.reshape(C, N)

    onehot = pl.pallas_call(
        _router_kernel,
        out_shape=jax.ShapeDtypeStruct((C, G), jnp.float32),
    )(y, fc1_w.T, fc1_b.reshape(1, N), fc2_w.T, fc2_b.reshape(1, G))

    out = pl.pallas_call(
        _norm_kernel,
        grid=(N,),
        in_specs=[
            pl.BlockSpec((1, C, H, W), lambda n: (n, 0, 0, 0)),
            pl.BlockSpec((C, G), lambda n: (0, 0)),
            pl.BlockSpec((C, 1), lambda n: (0, 0)),
            pl.BlockSpec((C, 1), lambda n: (0, 0)),
        ],
        out_specs=pl.BlockSpec((1, C, H, W), lambda n: (n, 0, 0, 0)),
        out_shape=jax.ShapeDtypeStruct((N, C, H, W), jnp.float32),
        compiler_params=pltpu.CompilerParams(
            dimension_semantics=("parallel",)),
    )(x, onehot, gamma.reshape(C, 1), beta.reshape(C, 1))
    return out


# speedup vs baseline: 1.0600x; 1.0146x over previous
import functools

import jax
import jax.numpy as jnp
from jax.experimental import pallas as pl
from jax.experimental.pallas import tpu as pltpu

N, C, H, W = 32, 192, 56, 56
G = 8
EPS = 1e-05
CB = 4
HW = H * W
WP = 64


def _conv_pool_kernel(x_ref, w2_ref, y_ref):
    xp = x_ref[...]
    dims = (((1,), (0,)), ((), ()))
    z = jax.lax.dot_general(
        xp[:, 0:H].reshape(CB * H * WP, N), w2_ref[0:32],
        dims, preferred_element_type=jnp.float32)
    z = z + jax.lax.dot_general(
        xp[:, 1:1 + H].reshape(CB * H * WP, N), w2_ref[32:64],
        dims, preferred_element_type=jnp.float32)
    z = z + jax.lax.dot_general(
        xp[:, 2:2 + H].reshape(CB * H * WP, N), w2_ref[64:96],
        dims, preferred_element_type=jnp.float32)
    z = z.reshape(CB, H, WP, 96)
    acc = (z[:, :, 0:W, 0:32]
           + z[:, :, 1:W + 1, 32:64]
           + z[:, :, 2:W + 2, 64:96])
    r = jnp.maximum(acc, 0.0)
    y_ref[0] = jnp.sum(r, axis=(1, 2)) * jnp.float32(1.0 / HW)


def _router_kernel(y_ref, w1t_ref, b1_ref, w2t_ref, b2_ref, oh_ref):
    y = y_ref[...]
    h1 = jnp.dot(y, w1t_ref[...],
                 preferred_element_type=jnp.float32) + b1_ref[...]
    h2 = jnp.dot(h1, w2t_ref[...],
                 preferred_element_type=jnp.float32) + b2_ref[...]
    m = jnp.max(h2, axis=1, keepdims=True)
    idx = jax.lax.broadcasted_iota(jnp.int32, (C, G), 1)
    first = jnp.min(jnp.where(h2 >= m, idx, G), axis=1, keepdims=True)
    oh_ref[...] = (idx == first).astype(jnp.float32)


def _norm_kernel(x_ref, oh_ref, g_ref, b_ref, o_ref):
    xb = x_ref[0]
    oh = oh_ref[...]
    rs = jnp.sum(xb, axis=(1, 2)).reshape(C, 1)
    rs2 = jnp.sum(xb * xb, axis=(1, 2)).reshape(C, 1)
    ones = jnp.ones((C, 1), jnp.float32)
    cdot = functools.partial(jax.lax.dot_general,
                             dimension_numbers=(((0,), (0,)), ((), ())),
                             preferred_element_type=jnp.float32)
    gsum = cdot(oh, rs)
    gsum2 = cdot(oh, rs2)
    n = cdot(oh, ones) * jnp.float32(HW)
    mean = gsum / jnp.maximum(n, 1.0)
    var = (gsum2 - n * mean * mean) / jnp.maximum(n - 1.0, 1.0)
    inv = 1.0 / jnp.sqrt(var + EPS)
    mean_c = jnp.dot(oh, mean, preferred_element_type=jnp.float32)
    inv_c = jnp.dot(oh, inv, preferred_element_type=jnp.float32)
    a = (inv_c * g_ref[...])[:, :, None]
    o_ref[0] = (xb - mean_c[:, :, None]) * a + b_ref[...][:, :, None]


def kernel(x, conv_w, fc1_w, fc1_b, fc2_w, fc2_b, gamma, beta):
    w2 = jnp.transpose(conv_w, (2, 1, 3, 0)).reshape(96, 96)
    xt = jnp.transpose(x, (1, 2, 3, 0))
    xtp = jnp.pad(xt, ((0, 0), (1, 1), (1, WP - W - 1), (0, 0)))

    y = pl.pallas_call(
        _conv_pool_kernel,
        grid=(C // CB,),
        in_specs=[
            pl.BlockSpec((CB, H + 2, WP, N), lambda i: (i, 0, 0, 0)),
            pl.BlockSpec((96, 96), lambda i: (0, 0)),
        ],
        out_specs=pl.BlockSpec((1, CB, N), lambda i: (i, 0, 0)),
        out_shape=jax.ShapeDtypeStruct((C // CB, CB, N), jnp.float32),
        compiler_params=pltpu.CompilerParams(
            dimension_semantics=("arbitrary",)),
    )(xtp, w2)
    y = y.reshape(C, N)

    onehot = pl.pallas_call(
        _router_kernel,
        out_shape=jax.ShapeDtypeStruct((C, G), jnp.float32),
    )(y, fc1_w.T, fc1_b.reshape(1, N), fc2_w.T, fc2_b.reshape(1, G))

    out = pl.pallas_call(
        _norm_kernel,
        grid=(N,),
        in_specs=[
            pl.BlockSpec((1, C, H, W), lambda n: (n, 0, 0, 0)),
            pl.BlockSpec((C, G), lambda n: (0, 0)),
            pl.BlockSpec((C, 1), lambda n: (0, 0)),
            pl.BlockSpec((C, 1), lambda n: (0, 0)),
        ],
        out_specs=pl.BlockSpec((1, C, H, W), lambda n: (n, 0, 0, 0)),
        out_shape=jax.ShapeDtypeStruct((N, C, H, W), jnp.float32),
        compiler_params=pltpu.CompilerParams(
            dimension_semantics=("parallel",)),
    )(x, onehot, gamma.reshape(C, 1), beta.reshape(C, 1))
    return out


# speedup vs baseline: 1.3873x; 1.3087x over previous
import functools

import jax
import jax.numpy as jnp
from jax.experimental import pallas as pl
from jax.experimental.pallas import tpu as pltpu

N, C, H, W = 32, 192, 56, 56
G = 8
EPS = 1e-05
CB = 4
HW = H * W
WP = 64


def _conv_pool_kernel(x_ref, w2_ref, y_ref):
    xt = x_ref[...]
    zw0 = jnp.zeros((CB, H, 1, N), xt.dtype)
    zw1 = jnp.zeros((CB, H, WP - W - 1, N), xt.dtype)
    xq = jnp.concatenate([zw0, xt, zw1], axis=2)
    zh = jnp.zeros((CB, 1, WP, N), xt.dtype)
    xp = jnp.concatenate([zh, xq, zh], axis=1)
    dims = (((1,), (0,)), ((), ()))
    z = jax.lax.dot_general(
        xp[:, 0:H].reshape(CB * H * WP, N), w2_ref[0:32],
        dims, preferred_element_type=jnp.float32)
    z = z + jax.lax.dot_general(
        xp[:, 1:1 + H].reshape(CB * H * WP, N), w2_ref[32:64],
        dims, preferred_element_type=jnp.float32)
    z = z + jax.lax.dot_general(
        xp[:, 2:2 + H].reshape(CB * H * WP, N), w2_ref[64:96],
        dims, preferred_element_type=jnp.float32)
    z = z.reshape(CB, H, WP, 96)
    acc = (z[:, :, 0:W, 0:32]
           + z[:, :, 1:W + 1, 32:64]
           + z[:, :, 2:W + 2, 64:96])
    r = jnp.maximum(acc, 0.0)
    y_ref[0] = jnp.sum(r, axis=(1, 2)) * jnp.float32(1.0 / HW)


def _router_kernel(y_ref, w1t_ref, b1_ref, w2t_ref, b2_ref, oh_ref):
    y = y_ref[...]
    h1 = jnp.dot(y, w1t_ref[...],
                 preferred_element_type=jnp.float32) + b1_ref[...]
    h2 = jnp.dot(h1, w2t_ref[...],
                 preferred_element_type=jnp.float32) + b2_ref[...]
    m = jnp.max(h2, axis=1, keepdims=True)
    idx = jax.lax.broadcasted_iota(jnp.int32, (C, G), 1)
    first = jnp.min(jnp.where(h2 >= m, idx, G), axis=1, keepdims=True)
    oh_ref[...] = (idx == first).astype(jnp.float32)


def _norm_kernel(x_ref, oh_ref, g_ref, b_ref, o_ref):
    xb = x_ref[0]
    oh = oh_ref[...]
    rs = jnp.sum(xb, axis=(1, 2)).reshape(C, 1)
    rs2 = jnp.sum(xb * xb, axis=(1, 2)).reshape(C, 1)
    ones = jnp.ones((C, 1), jnp.float32)
    cdot = functools.partial(jax.lax.dot_general,
                             dimension_numbers=(((0,), (0,)), ((), ())),
                             preferred_element_type=jnp.float32)
    gsum = cdot(oh, rs)
    gsum2 = cdot(oh, rs2)
    n = cdot(oh, ones) * jnp.float32(HW)
    mean = gsum / jnp.maximum(n, 1.0)
    var = (gsum2 - n * mean * mean) / jnp.maximum(n - 1.0, 1.0)
    inv = 1.0 / jnp.sqrt(var + EPS)
    mean_c = jnp.dot(oh, mean, preferred_element_type=jnp.float32)
    inv_c = jnp.dot(oh, inv, preferred_element_type=jnp.float32)
    a = (inv_c * g_ref[...])[:, :, None]
    o_ref[0] = (xb - mean_c[:, :, None]) * a + b_ref[...][:, :, None]


def kernel(x, conv_w, fc1_w, fc1_b, fc2_w, fc2_b, gamma, beta):
    w2 = jnp.transpose(conv_w, (2, 1, 3, 0)).reshape(96, 96)
    xt = jnp.transpose(x, (1, 2, 3, 0))

    y = pl.pallas_call(
        _conv_pool_kernel,
        grid=(C // CB,),
        in_specs=[
            pl.BlockSpec((CB, H, W, N), lambda i: (i, 0, 0, 0)),
            pl.BlockSpec((96, 96), lambda i: (0, 0)),
        ],
        out_specs=pl.BlockSpec((1, CB, N), lambda i: (i, 0, 0)),
        out_shape=jax.ShapeDtypeStruct((C // CB, CB, N), jnp.float32),
        compiler_params=pltpu.CompilerParams(
            dimension_semantics=("arbitrary",)),
    )(xt, w2)
    y = y.reshape(C, N)

    onehot = pl.pallas_call(
        _router_kernel,
        out_shape=jax.ShapeDtypeStruct((C, G), jnp.float32),
    )(y, fc1_w.T, fc1_b.reshape(1, N), fc2_w.T, fc2_b.reshape(1, G))

    out = pl.pallas_call(
        _norm_kernel,
        grid=(N,),
        in_specs=[
            pl.BlockSpec((1, C, H, W), lambda n: (n, 0, 0, 0)),
            pl.BlockSpec((C, G), lambda n: (0, 0)),
            pl.BlockSpec((C, 1), lambda n: (0, 0)),
            pl.BlockSpec((C, 1), lambda n: (0, 0)),
        ],
        out_specs=pl.BlockSpec((1, C, H, W), lambda n: (n, 0, 0, 0)),
        out_shape=jax.ShapeDtypeStruct((N, C, H, W), jnp.float32),
        compiler_params=pltpu.CompilerParams(
            dimension_semantics=("parallel",)),
    )(x, onehot, gamma.reshape(C, 1), beta.reshape(C, 1))
    return out


# speedup vs baseline: 1.3914x; 1.0030x over previous
import functools

import jax
import jax.numpy as jnp
from jax.experimental import pallas as pl
from jax.experimental.pallas import tpu as pltpu

N, C, H, W = 32, 192, 56, 56
G = 8
EPS = 1e-05
CB = 8
HW = H * W
WP = 64


def _conv_pool_kernel(x_ref, w2_ref, y_ref):
    xt = x_ref[...]
    zw0 = jnp.zeros((CB, H, 1, N), xt.dtype)
    zw1 = jnp.zeros((CB, H, WP - W - 1, N), xt.dtype)
    xq = jnp.concatenate([zw0, xt, zw1], axis=2)
    zh = jnp.zeros((CB, 1, WP, N), xt.dtype)
    xp = jnp.concatenate([zh, xq, zh], axis=1)
    dims = (((1,), (0,)), ((), ()))
    z = jax.lax.dot_general(
        xp[:, 0:H].reshape(CB * H * WP, N), w2_ref[0:32],
        dims, preferred_element_type=jnp.float32)
    z = z + jax.lax.dot_general(
        xp[:, 1:1 + H].reshape(CB * H * WP, N), w2_ref[32:64],
        dims, preferred_element_type=jnp.float32)
    z = z + jax.lax.dot_general(
        xp[:, 2:2 + H].reshape(CB * H * WP, N), w2_ref[64:96],
        dims, preferred_element_type=jnp.float32)
    z = z.reshape(CB, H, WP, 96)
    acc = (z[:, :, 0:W, 0:32]
           + z[:, :, 1:W + 1, 32:64]
           + z[:, :, 2:W + 2, 64:96])
    r = jnp.maximum(acc, 0.0)
    y_ref[0] = jnp.sum(r, axis=(1, 2)) * jnp.float32(1.0 / HW)


def _router_kernel(y_ref, w1t_ref, b1_ref, w2t_ref, b2_ref, oh_ref):
    y = y_ref[...]
    h1 = jnp.dot(y, w1t_ref[...],
                 preferred_element_type=jnp.float32) + b1_ref[...]
    h2 = jnp.dot(h1, w2t_ref[...],
                 preferred_element_type=jnp.float32) + b2_ref[...]
    m = jnp.max(h2, axis=1, keepdims=True)
    idx = jax.lax.broadcasted_iota(jnp.int32, (C, G), 1)
    first = jnp.min(jnp.where(h2 >= m, idx, G), axis=1, keepdims=True)
    oh_ref[...] = (idx == first).astype(jnp.float32)


def _norm_kernel(x_ref, oh_ref, g_ref, b_ref, o_ref):
    xb = x_ref[0]
    oh = oh_ref[...]
    rs = jnp.sum(xb, axis=(1, 2)).reshape(C, 1)
    rs2 = jnp.sum(xb * xb, axis=(1, 2)).reshape(C, 1)
    ones = jnp.ones((C, 1), jnp.float32)
    cdot = functools.partial(jax.lax.dot_general,
                             dimension_numbers=(((0,), (0,)), ((), ())),
                             preferred_element_type=jnp.float32)
    gsum = cdot(oh, rs)
    gsum2 = cdot(oh, rs2)
    n = cdot(oh, ones) * jnp.float32(HW)
    mean = gsum / jnp.maximum(n, 1.0)
    var = (gsum2 - n * mean * mean) / jnp.maximum(n - 1.0, 1.0)
    inv = 1.0 / jnp.sqrt(var + EPS)
    mean_c = jnp.dot(oh, mean, preferred_element_type=jnp.float32)
    inv_c = jnp.dot(oh, inv, preferred_element_type=jnp.float32)
    a = (inv_c * g_ref[...])[:, :, None]
    o_ref[0] = (xb - mean_c[:, :, None]) * a + b_ref[...][:, :, None]


def kernel(x, conv_w, fc1_w, fc1_b, fc2_w, fc2_b, gamma, beta):
    w2 = jnp.transpose(conv_w, (2, 1, 3, 0)).reshape(96, 96)
    xt = jnp.transpose(x, (1, 2, 3, 0))

    y = pl.pallas_call(
        _conv_pool_kernel,
        grid=(C // CB,),
        in_specs=[
            pl.BlockSpec((CB, H, W, N), lambda i: (i, 0, 0, 0)),
            pl.BlockSpec((96, 96), lambda i: (0, 0)),
        ],
        out_specs=pl.BlockSpec((1, CB, N), lambda i: (i, 0, 0)),
        out_shape=jax.ShapeDtypeStruct((C // CB, CB, N), jnp.float32),
        compiler_params=pltpu.CompilerParams(
            dimension_semantics=("arbitrary",)),
    )(xt, w2)
    y = y.reshape(C, N)

    onehot = pl.pallas_call(
        _router_kernel,
        out_shape=jax.ShapeDtypeStruct((C, G), jnp.float32),
    )(y, fc1_w.T, fc1_b.reshape(1, N), fc2_w.T, fc2_b.reshape(1, G))

    out = pl.pallas_call(
        _norm_kernel,
        grid=(N,),
        in_specs=[
            pl.BlockSpec((1, C, H, W), lambda n: (n, 0, 0, 0)),
            pl.BlockSpec((C, G), lambda n: (0, 0)),
            pl.BlockSpec((C, 1), lambda n: (0, 0)),
            pl.BlockSpec((C, 1), lambda n: (0, 0)),
        ],
        out_specs=pl.BlockSpec((1, C, H, W), lambda n: (n, 0, 0, 0)),
        out_shape=jax.ShapeDtypeStruct((N, C, H, W), jnp.float32),
        compiler_params=pltpu.CompilerParams(
            dimension_semantics=("parallel",)),
    )(x, onehot, gamma.reshape(C, 1), beta.reshape(C, 1))
    return out


# speedup vs baseline: 1.4050x; 1.0098x over previous
import functools

import jax
import jax.numpy as jnp
from jax.experimental import pallas as pl
from jax.experimental.pallas import tpu as pltpu

N, C, H, W = 32, 192, 56, 56
G = 8
EPS = 1e-05
CB = 8
HW = H * W
WP = 64


def _conv_pool_kernel(x_ref, w2_ref, y_ref):
    xt = x_ref[...]
    zw0 = jnp.zeros((CB, H, 1, N), xt.dtype)
    zw1 = jnp.zeros((CB, H, WP - W - 1, N), xt.dtype)
    xq = jnp.concatenate([zw0, xt, zw1], axis=2)
    zh = jnp.zeros((CB, 1, WP, N), xt.dtype)
    xp = jnp.concatenate([zh, xq, zh], axis=1)
    dims = (((1,), (0,)), ((), ()))
    z = jax.lax.dot_general(
        xp[:, 0:H].reshape(CB * H * WP, N), w2_ref[0:32],
        dims, preferred_element_type=jnp.float32)
    z = z + jax.lax.dot_general(
        xp[:, 1:1 + H].reshape(CB * H * WP, N), w2_ref[32:64],
        dims, preferred_element_type=jnp.float32)
    z = z + jax.lax.dot_general(
        xp[:, 2:2 + H].reshape(CB * H * WP, N), w2_ref[64:96],
        dims, preferred_element_type=jnp.float32)
    z = z.reshape(CB, H, WP, 96)
    acc = (z[:, :, 0:W, 0:32]
           + z[:, :, 1:W + 1, 32:64]
           + z[:, :, 2:W + 2, 64:96])
    r = jnp.maximum(acc, 0.0)
    y_ref[0] = jnp.sum(r, axis=(1, 2)) * jnp.float32(1.0 / HW)


def _router_kernel(y_ref, w1t_ref, b1_ref, w2t_ref, b2_ref, oh_ref):
    y = y_ref[...]
    h1 = jnp.dot(y, w1t_ref[...],
                 preferred_element_type=jnp.float32) + b1_ref[...]
    h2 = jnp.dot(h1, w2t_ref[...],
                 preferred_element_type=jnp.float32) + b2_ref[...]
    m = jnp.max(h2, axis=1, keepdims=True)
    idx = jax.lax.broadcasted_iota(jnp.int32, (C, G), 1)
    first = jnp.min(jnp.where(h2 >= m, idx, G), axis=1, keepdims=True)
    oh_ref[...] = (idx == first).astype(jnp.float32)


def _norm_kernel(x_ref, oh_ref, g_ref, b_ref, o_ref):
    xb = x_ref[0]
    oh = oh_ref[...]
    rs = jnp.sum(xb, axis=(1, 2)).reshape(C, 1)
    rs2 = jnp.sum(xb * xb, axis=(1, 2)).reshape(C, 1)
    ones = jnp.ones((C, 1), jnp.float32)
    cdot = functools.partial(jax.lax.dot_general,
                             dimension_numbers=(((0,), (0,)), ((), ())),
                             preferred_element_type=jnp.float32)
    gsum = cdot(oh, rs)
    gsum2 = cdot(oh, rs2)
    n = cdot(oh, ones) * jnp.float32(HW)
    mean = gsum / jnp.maximum(n, 1.0)
    var = (gsum2 - n * mean * mean) / jnp.maximum(n - 1.0, 1.0)
    inv = 1.0 / jnp.sqrt(var + EPS)
    mean_c = jnp.dot(oh, mean, preferred_element_type=jnp.float32)
    inv_c = jnp.dot(oh, inv, preferred_element_type=jnp.float32)
    a = inv_c * g_ref[...]
    b = b_ref[...] - mean_c * a
    o_ref[0] = xb * a[:, :, None] + b[:, :, None]


def kernel(x, conv_w, fc1_w, fc1_b, fc2_w, fc2_b, gamma, beta):
    w2 = jnp.transpose(conv_w, (2, 1, 3, 0)).reshape(96, 96)
    xt = jnp.transpose(x, (1, 2, 3, 0))

    y = pl.pallas_call(
        _conv_pool_kernel,
        grid=(C // CB,),
        in_specs=[
            pl.BlockSpec((CB, H, W, N), lambda i: (i, 0, 0, 0)),
            pl.BlockSpec((96, 96), lambda i: (0, 0)),
        ],
        out_specs=pl.BlockSpec((1, CB, N), lambda i: (i, 0, 0)),
        out_shape=jax.ShapeDtypeStruct((C // CB, CB, N), jnp.float32),
        compiler_params=pltpu.CompilerParams(
            dimension_semantics=("parallel",)),
    )(xt, w2)
    y = y.reshape(C, N)

    onehot = pl.pallas_call(
        _router_kernel,
        out_shape=jax.ShapeDtypeStruct((C, G), jnp.float32),
    )(y, fc1_w.T, fc1_b.reshape(1, N), fc2_w.T, fc2_b.reshape(1, G))

    out = pl.pallas_call(
        _norm_kernel,
        grid=(N,),
        in_specs=[
            pl.BlockSpec((1, C, H, W), lambda n: (n, 0, 0, 0)),
            pl.BlockSpec((C, G), lambda n: (0, 0)),
            pl.BlockSpec((C, 1), lambda n: (0, 0)),
            pl.BlockSpec((C, 1), lambda n: (0, 0)),
        ],
        out_specs=pl.BlockSpec((1, C, H, W), lambda n: (n, 0, 0, 0)),
        out_shape=jax.ShapeDtypeStruct((N, C, H, W), jnp.float32),
        compiler_params=pltpu.CompilerParams(
            dimension_semantics=("parallel",)),
    )(x, onehot, gamma.reshape(C, 1), beta.reshape(C, 1))
    return out


# speedup vs baseline: 1.4580x; 1.0377x over previous
import functools

import jax
import jax.numpy as jnp
from jax.experimental import pallas as pl
from jax.experimental.pallas import tpu as pltpu

N, C, H, W = 32, 192, 56, 56
G = 8
EPS = 1e-05
CB = 8
HW = H * W
WP = 64


def _conv_pool_kernel(x_ref, w2_ref, y_ref):
    xt = x_ref[...]
    zw0 = jnp.zeros((CB, H, 1, N), xt.dtype)
    zw1 = jnp.zeros((CB, H, WP - W - 1, N), xt.dtype)
    xq = jnp.concatenate([zw0, xt, zw1], axis=2)
    zh = jnp.zeros((CB, 1, WP, N), xt.dtype)
    xp = jnp.concatenate([zh, xq, zh], axis=1)
    dims = (((1,), (0,)), ((), ()))
    z = jax.lax.dot_general(
        xp[:, 0:H].reshape(CB * H * WP, N), w2_ref[0:32],
        dims, preferred_element_type=jnp.float32)
    z = z + jax.lax.dot_general(
        xp[:, 1:1 + H].reshape(CB * H * WP, N), w2_ref[32:64],
        dims, preferred_element_type=jnp.float32)
    z = z + jax.lax.dot_general(
        xp[:, 2:2 + H].reshape(CB * H * WP, N), w2_ref[64:96],
        dims, preferred_element_type=jnp.float32)
    z = z.reshape(CB, H, WP, 96)
    acc = (z[:, :, 0:W, 0:32]
           + z[:, :, 1:W + 1, 32:64]
           + z[:, :, 2:W + 2, 64:96])
    r = jnp.maximum(acc, 0.0)
    y_ref[0] = jnp.sum(r, axis=(1, 2)) * jnp.float32(1.0 / HW)


def _router_kernel(y_ref, w1t_ref, b1_ref, w2t_ref, b2_ref, oh_ref):
    y = y_ref[...]
    h1 = jnp.dot(y, w1t_ref[...],
                 preferred_element_type=jnp.float32) + b1_ref[...]
    h2 = jnp.dot(h1, w2t_ref[...],
                 preferred_element_type=jnp.float32) + b2_ref[...]
    m = jnp.max(h2, axis=1, keepdims=True)
    idx = jax.lax.broadcasted_iota(jnp.int32, (C, G), 1)
    first = jnp.min(jnp.where(h2 >= m, idx, G), axis=1, keepdims=True)
    oh_ref[...] = (idx == first).astype(jnp.float32)


def _norm_kernel(x_ref, oh_ref, g_ref, b_ref, o_ref):
    xb = x_ref[0]
    oh = oh_ref[...]
    rs = jnp.sum(xb, axis=1, keepdims=True)
    rs2 = jnp.sum(xb * xb, axis=1, keepdims=True)
    ones = jnp.ones((C, 1), jnp.float32)
    cdot = functools.partial(jax.lax.dot_general,
                             dimension_numbers=(((0,), (0,)), ((), ())),
                             preferred_element_type=jnp.float32)
    gsum = cdot(oh, rs)
    gsum2 = cdot(oh, rs2)
    n = cdot(oh, ones) * jnp.float32(HW)
    mean = gsum / jnp.maximum(n, 1.0)
    var = (gsum2 - n * mean * mean) / jnp.maximum(n - 1.0, 1.0)
    inv = 1.0 / jnp.sqrt(var + EPS)
    mean_c = jnp.dot(oh, mean, preferred_element_type=jnp.float32)
    inv_c = jnp.dot(oh, inv, preferred_element_type=jnp.float32)
    a = inv_c * g_ref[...]
    b = b_ref[...] - mean_c * a
    o_ref[0] = (xb * a + b).reshape(C, H, W)


def kernel(x, conv_w, fc1_w, fc1_b, fc2_w, fc2_b, gamma, beta):
    w2 = jnp.transpose(conv_w, (2, 1, 3, 0)).reshape(96, 96)
    xt = jnp.transpose(x, (1, 2, 3, 0))

    y = pl.pallas_call(
        _conv_pool_kernel,
        grid=(C // CB,),
        in_specs=[
            pl.BlockSpec((CB, H, W, N), lambda i: (i, 0, 0, 0)),
            pl.BlockSpec((96, 96), lambda i: (0, 0)),
        ],
        out_specs=pl.BlockSpec((1, CB, N), lambda i: (i, 0, 0)),
        out_shape=jax.ShapeDtypeStruct((C // CB, CB, N), jnp.float32),
        compiler_params=pltpu.CompilerParams(
            dimension_semantics=("parallel",)),
    )(xt, w2)
    y = y.reshape(C, N)

    onehot = pl.pallas_call(
        _router_kernel,
        out_shape=jax.ShapeDtypeStruct((C, G), jnp.float32),
    )(y, fc1_w.T, fc1_b.reshape(1, N), fc2_w.T, fc2_b.reshape(1, G))

    out = pl.pallas_call(
        _norm_kernel,
        grid=(N,),
        in_specs=[
            pl.BlockSpec((1, C, HW), lambda n: (n, 0, 0)),
            pl.BlockSpec((C, G), lambda n: (0, 0)),
            pl.BlockSpec((C, 1), lambda n: (0, 0)),
            pl.BlockSpec((C, 1), lambda n: (0, 0)),
        ],
        out_specs=pl.BlockSpec((1, C, H, W), lambda n: (n, 0, 0, 0)),
        out_shape=jax.ShapeDtypeStruct((N, C, H, W), jnp.float32),
        compiler_params=pltpu.CompilerParams(
            dimension_semantics=("parallel",)),
    )(x.reshape(N, C, HW), onehot, gamma.reshape(C, 1), beta.reshape(C, 1))
    return out
